# baseline (speedup 1.0000x reference)
import functools

import jax
import jax.numpy as jnp
import numpy as np
from jax import lax
from jax.experimental import pallas as pl
from jax.experimental.pallas import tpu as pltpu
from jax.experimental.pallas import tpu_sc as plsc

N_NODES = 10000
D = 128
E_EDGES = 320000
NC, NS = 2, 16
NW = NC * NS
CHUNK = 128
NCHUNK = 160
SEG = 32
NSEG = NCHUNK // SEG
E_PER = NCHUNK * CHUNK
EC_PAD = NW * E_PER
N_PAD = 10240
ROWS_PER_TILE = N_PAD // NS
LANES = 16

DP = D // 2


def _sc_body(h_hbm, row_hbm, col_hbm, w_hbm, out_hbm,
             row_v, col_v, w_v, gbuf, gbuf1, sbuf, acc, sem, sem1):
    c = lax.axis_index("c")
    s = lax.axis_index("s")
    wid = c * NS + s

    zero = jnp.zeros((LANES,), jnp.float32)

    def zrow(i, carry):
        for j in range(D // LANES):
            sbuf[i, pl.ds(j * LANES, LANES)] = zero
        return carry

    lax.fori_loop(0, CHUNK, zrow, 0)
    for z in range(ROWS_PER_TILE // CHUNK):
        pltpu.sync_copy(
            sbuf, acc.at[pl.ds(s * ROWS_PER_TILE + z * CHUNK, CHUNK)])
    plsc.subcore_barrier()

    hi_mask = jnp.full((LANES,), -65536, jnp.int32)

    def scale(gb, k):
        for b in range(CHUNK // LANES):
            wv16 = w_v[k, pl.ds(b * LANES, LANES)]
            for l in range(LANES):
                wv = jnp.full((LANES,), wv16[l], jnp.float32)
                r = b * LANES + l
                for j in range(DP // LANES):
                    u = gb[r, pl.ds(j * LANES, LANES)]
                    lo = plsc.bitcast(u << 16, jnp.float32)
                    hi = plsc.bitcast(u & hi_mask, jnp.float32)
                    sbuf[r, pl.ds(j * LANES, LANES)] = lo * wv
                    sbuf[r, pl.ds(DP + j * LANES, LANES)] = hi * wv

    def seg_body(g, carry):
        pltpu.sync_copy(row_hbm.at[wid, pl.ds(g * SEG, SEG)], row_v)
        pltpu.sync_copy(col_hbm.at[wid, pl.ds(g * SEG, SEG)], col_v)
        pltpu.sync_copy(w_hbm.at[wid, pl.ds(g * SEG, SEG)], w_v)

        pltpu.async_copy(h_hbm.at[col_v.at[0]], gbuf, sem)

        def pair_body(k2, inner):
            kk0 = k2 * 2
            pltpu.make_async_copy(h_hbm.at[col_v.at[kk0]], gbuf, sem).wait()
            pltpu.async_copy(h_hbm.at[col_v.at[kk0 + 1]], gbuf1, sem1)
            scale(gbuf, kk0)
            pltpu.sync_copy(sbuf, acc.at[row_v.at[kk0]], add=True)

            pltpu.make_async_copy(
                h_hbm.at[col_v.at[kk0 + 1]], gbuf1, sem1).wait()

            @pl.when(kk0 + 2 < SEG)
            def _():
                pltpu.async_copy(h_hbm.at[col_v.at[kk0 + 2]], gbuf, sem)
            scale(gbuf1, kk0 + 1)
            pltpu.sync_copy(sbuf, acc.at[row_v.at[kk0 + 1]], add=True)
            return inner

        lax.fori_loop(0, SEG // 2, pair_body, 0)
        return carry

    lax.fori_loop(0, NSEG, seg_body, 0)

    plsc.subcore_barrier()
    pltpu.sync_copy(acc.at[pl.ds(s * ROWS_PER_TILE, ROWS_PER_TILE)],
                    out_hbm.at[c, pl.ds(s * ROWS_PER_TILE, ROWS_PER_TILE)])


_sc_aggregate = functools.partial(
    pl.kernel,
    out_type=jax.ShapeDtypeStruct((NC, N_PAD, D), jnp.float32),
    mesh=plsc.VectorSubcoreMesh(core_axis_name="c", subcore_axis_name="s"),
    compiler_params=pltpu.CompilerParams(
        needs_layout_passes=False, use_tc_tiling_on_sc=False),
    scratch_types=[
        pltpu.VMEM((SEG, CHUNK), jnp.int32),
        pltpu.VMEM((SEG, CHUNK), jnp.int32),
        pltpu.VMEM((SEG, CHUNK), jnp.float32),
        pltpu.VMEM((CHUNK, DP), jnp.int32),
        pltpu.VMEM((CHUNK, DP), jnp.int32),
        pltpu.VMEM((CHUNK, D), jnp.float32),
        pltpu.VMEM_SHARED((N_PAD, D), jnp.float32),
        pltpu.SemaphoreType.DMA,
        pltpu.SemaphoreType.DMA,
    ],
)(_sc_body)


BM = 2000


def _round_bf16_bits(v):
    bits = lax.bitcast_convert_type(v, jnp.int32)
    rounded = (bits + 0x7FFF + ((bits >> 16) & 1)) >> 16
    return rounded & 0xFFFF


def _mm_body(x_ref, w1_ref, b1_ref, w2_ref, b2_ref, o_ref):
    xb = x_ref[...]
    dims = (((1,), (1,)), ((), ()))
    h1 = lax.dot_general(
        xb, w1_ref[...], dims, preferred_element_type=jnp.float32) + b1_ref[...]
    h2 = lax.dot_general(
        xb, w2_ref[...], dims, preferred_element_type=jnp.float32) + b2_ref[...]
    for idx, h in ((0, h1), (1, h2)):
        lo = _round_bf16_bits(h[:, :DP])
        hi = _round_bf16_bits(h[:, DP:])
        o_ref[idx] = lo | (hi << 16)


def _hidden(x, W1, b1, W2, b2):
    h = pl.pallas_call(
        _mm_body,
        grid=(N_NODES // BM,),
        in_specs=[
            pl.BlockSpec((BM, D), lambda i: (i, 0)),
            pl.BlockSpec((D, D), lambda i: (0, 0)),
            pl.BlockSpec((1, D), lambda i: (0, 0)),
            pl.BlockSpec((D, D), lambda i: (0, 0)),
            pl.BlockSpec((1, D), lambda i: (0, 0)),
        ],
        out_specs=pl.BlockSpec((2, BM, DP), lambda i: (0, i, 0)),
        out_shape=jax.ShapeDtypeStruct((2, N_NODES, DP), jnp.int32),
    )(x, W1, b1.reshape(1, D), W2, b2.reshape(1, D))
    return h.reshape(2 * N_NODES, DP)


def _add_body(p_ref, o_ref):
    o_ref[...] = p_ref[0] + p_ref[1]


def _final_add(partials):
    return pl.pallas_call(
        _add_body,
        grid=(N_NODES // BM,),
        in_specs=[pl.BlockSpec((2, BM, D), lambda i: (0, i, 0))],
        out_specs=pl.BlockSpec((BM, D), lambda i: (i, 0)),
        out_shape=jax.ShapeDtypeStruct((N_NODES, D), jnp.float32),
    )(partials)


def kernel(x, edge_index1, edge_weight1, edge_index2, edge_weight2,
           W1, b1, W2, b2):
    pad = EC_PAD - 2 * E_EDGES
    spread = jnp.arange(pad, dtype=jnp.int32)
    rows = jnp.concatenate(
        [edge_index1[0], edge_index2[0],
         spread % N_PAD]).reshape(NW, NCHUNK, CHUNK)
    cols = jnp.concatenate(
        [edge_index1[1], edge_index2[1] + N_NODES,
         spread % (2 * N_NODES)]).reshape(NW, NCHUNK, CHUNK)
    w = jnp.concatenate(
        [edge_weight1, edge_weight2,
         jnp.zeros((pad,), jnp.float32)]).reshape(NW, NCHUNK, CHUNK)
    H = _hidden(x, W1, b1, W2, b2)
    partials = _sc_aggregate(H, rows, cols, w)
    return _final_add(partials)


# speedup vs baseline: 1.0297x; 1.0297x over previous
import functools

import jax
import jax.numpy as jnp
from jax import lax
from jax.experimental import pallas as pl
from jax.experimental.pallas import tpu as pltpu
from jax.experimental.pallas import tpu_sc as plsc

N_NODES = 10000
D = 128
E_EDGES = 320000
NC, NS = 2, 16
NW = NC * NS
CHUNK = 128
NCHUNK = 160
SEG = 32
NSEG = NCHUNK // SEG
E_PER = NCHUNK * CHUNK
EC_PAD = NW * E_PER
N_PAD = 10240
ROWS_PER_TILE = N_PAD // NS
LANES = 16


def _sc_body(h_hbm, row_hbm, col_hbm, w_hbm, out_hbm,
             row_v, col_v, w_v, gbuf, gbuf1, acc, sem, sem1, ssem, ssem1):
    c = lax.axis_index("c")
    s = lax.axis_index("s")
    wid = c * NS + s

    zero = jnp.zeros((LANES,), jnp.float32)

    def zrow(i, carry):
        for j in range(D // LANES):
            gbuf[i, pl.ds(j * LANES, LANES)] = zero
        return carry

    lax.fori_loop(0, CHUNK, zrow, 0)
    for z in range(ROWS_PER_TILE // CHUNK):
        pltpu.sync_copy(
            gbuf, acc.at[pl.ds(s * ROWS_PER_TILE + z * CHUNK, CHUNK)])
    plsc.subcore_barrier()

    def scale(gb, k):
        for b in range(CHUNK // LANES):
            wv16 = w_v[k, pl.ds(b * LANES, LANES)]
            for l in range(LANES):
                wv = jnp.full((LANES,), wv16[l], jnp.float32)
                r = b * LANES + l
                for j in range(D // LANES):
                    gb[r, pl.ds(j * LANES, LANES)] = (
                        gb[r, pl.ds(j * LANES, LANES)] * wv)

    def drain_scatter(gb, sm):
        pltpu.make_async_copy(gb, acc.at[pl.ds(0, CHUNK)], sm).wait()

    def seg_body(g, carry):
        pltpu.sync_copy(row_hbm.at[wid, pl.ds(g * SEG, SEG)], row_v)
        pltpu.sync_copy(col_hbm.at[wid, pl.ds(g * SEG, SEG)], col_v)
        pltpu.sync_copy(w_hbm.at[wid, pl.ds(g * SEG, SEG)], w_v)

        pltpu.async_copy(h_hbm.at[col_v.at[0]], gbuf, sem)

        def pair_body(k2, inner):
            kk0 = k2 * 2
            pltpu.make_async_copy(h_hbm.at[col_v.at[kk0]], gbuf, sem).wait()

            @pl.when(k2 > 0)
            def _():
                drain_scatter(gbuf1, ssem1)
            pltpu.async_copy(h_hbm.at[col_v.at[kk0 + 1]], gbuf1, sem1)
            scale(gbuf, kk0)
            pltpu.async_copy(gbuf, acc.at[row_v.at[kk0]], ssem, add=True)

            pltpu.make_async_copy(
                h_hbm.at[col_v.at[kk0 + 1]], gbuf1, sem1).wait()

            @pl.when(kk0 + 2 < SEG)
            def _():
                drain_scatter(gbuf, ssem)
                pltpu.async_copy(h_hbm.at[col_v.at[kk0 + 2]], gbuf, sem)
            scale(gbuf1, kk0 + 1)
            pltpu.async_copy(gbuf1, acc.at[row_v.at[kk0 + 1]], ssem1,
                             add=True)
            return inner

        lax.fori_loop(0, SEG // 2, pair_body, 0)
        drain_scatter(gbuf, ssem)
        drain_scatter(gbuf1, ssem1)
        return carry

    lax.fori_loop(0, NSEG, seg_body, 0)

    plsc.subcore_barrier()
    pltpu.sync_copy(acc.at[pl.ds(s * ROWS_PER_TILE, ROWS_PER_TILE)],
                    out_hbm.at[c, pl.ds(s * ROWS_PER_TILE, ROWS_PER_TILE)])


_sc_aggregate = functools.partial(
    pl.kernel,
    out_type=jax.ShapeDtypeStruct((NC, N_PAD, D), jnp.float32),
    mesh=plsc.VectorSubcoreMesh(core_axis_name="c", subcore_axis_name="s"),
    scratch_types=[
        pltpu.VMEM((SEG, CHUNK), jnp.int32),
        pltpu.VMEM((SEG, CHUNK), jnp.int32),
        pltpu.VMEM((SEG, CHUNK), jnp.float32),
        pltpu.VMEM((CHUNK, D), jnp.float32),
        pltpu.VMEM((CHUNK, D), jnp.float32),
        pltpu.VMEM_SHARED((N_PAD, D), jnp.float32),
        pltpu.SemaphoreType.DMA,
        pltpu.SemaphoreType.DMA,
        pltpu.SemaphoreType.DMA,
        pltpu.SemaphoreType.DMA,
    ],
)(_sc_body)


BM = 2000


def _mm_body(x_ref, w1_ref, b1_ref, w2_ref, b2_ref, o_ref):
    xb = x_ref[...]
    dims = (((1,), (1,)), ((), ()))
    o_ref[0] = lax.dot_general(
        xb, w1_ref[...], dims, preferred_element_type=jnp.float32) + b1_ref[...]
    o_ref[1] = lax.dot_general(
        xb, w2_ref[...], dims, preferred_element_type=jnp.float32) + b2_ref[...]


def _hidden(x, W1, b1, W2, b2):
    h = pl.pallas_call(
        _mm_body,
        grid=(N_NODES // BM,),
        in_specs=[
            pl.BlockSpec((BM, D), lambda i: (i, 0)),
            pl.BlockSpec((D, D), lambda i: (0, 0)),
            pl.BlockSpec((1, D), lambda i: (0, 0)),
            pl.BlockSpec((D, D), lambda i: (0, 0)),
            pl.BlockSpec((1, D), lambda i: (0, 0)),
        ],
        out_specs=pl.BlockSpec((2, BM, D), lambda i: (0, i, 0)),
        out_shape=jax.ShapeDtypeStruct((2, N_NODES, D), jnp.float32),
    )(x, W1, b1.reshape(1, D), W2, b2.reshape(1, D))
    return h.reshape(2 * N_NODES, D)


def _add_body(p_ref, o_ref):
    o_ref[...] = p_ref[0] + p_ref[1]


def _final_add(partials):
    return pl.pallas_call(
        _add_body,
        grid=(N_NODES // BM,),
        in_specs=[pl.BlockSpec((2, BM, D), lambda i: (0, i, 0))],
        out_specs=pl.BlockSpec((BM, D), lambda i: (i, 0)),
        out_shape=jax.ShapeDtypeStruct((N_NODES, D), jnp.float32),
    )(partials)


def kernel(x, edge_index1, edge_weight1, edge_index2, edge_weight2,
           W1, b1, W2, b2):
    pad = EC_PAD - 2 * E_EDGES
    spread = jnp.arange(pad, dtype=jnp.int32)
    rows = jnp.concatenate(
        [edge_index1[0], edge_index2[0],
         spread & 8191]).reshape(NW, NCHUNK, CHUNK)
    cols = jnp.concatenate(
        [edge_index1[1], edge_index2[1] + N_NODES,
         spread & 16383]).reshape(NW, NCHUNK, CHUNK)
    w = jnp.concatenate(
        [edge_weight1, edge_weight2,
         jnp.zeros((pad,), jnp.float32)]).reshape(NW, NCHUNK, CHUNK)
    H = _hidden(x, W1, b1, W2, b2)
    partials = _sc_aggregate(H, rows, cols, w)
    return _final_add(partials)
